# Initial kernel scaffold; baseline (speedup 1.0000x reference)
import jax
import jax.numpy as jnp
from jax.experimental import pallas as pl


def kernel(x, W):
    raise NotImplementedError("write your pallas kernel here")


# speedup vs baseline: 3.0258x; 3.0258x over previous
import jax
import jax.numpy as jnp
from jax.experimental import pallas as pl

DIM = 1024
SCALE = DIM ** (-0.5)


def _scale_kernel(w_ref, o_ref):
    o_ref[...] = w_ref[...] * SCALE


def kernel(x, W):
    n = x.shape[1]
    rows_per_block = 1024
    grid = (n // rows_per_block,)
    return pl.pallas_call(
        _scale_kernel,
        grid=grid,
        in_specs=[pl.BlockSpec((rows_per_block, DIM), lambda i: (i, 0))],
        out_specs=pl.BlockSpec((rows_per_block, DIM), lambda i: (i, 0)),
        out_shape=jax.ShapeDtypeStruct((n, DIM), W.dtype),
    )(W[:n])


# speedup vs baseline: 3.2875x; 1.0865x over previous
import jax
import jax.numpy as jnp
from jax.experimental import pallas as pl

DIM = 1024
SCALE = DIM ** (-0.5)


def _scale_kernel(w_ref, o_ref):
    o_ref[...] = w_ref[...] * SCALE


def kernel(x, W):
    n = x.shape[1]
    rows_per_block = 2048
    grid = (n // rows_per_block,)
    return pl.pallas_call(
        _scale_kernel,
        grid=grid,
        in_specs=[pl.BlockSpec((rows_per_block, DIM), lambda i: (i, 0))],
        out_specs=pl.BlockSpec((rows_per_block, DIM), lambda i: (i, 0)),
        out_shape=jax.ShapeDtypeStruct((n, DIM), W.dtype),
    )(W[:n])
